# baseline (speedup 1.0000x reference)
import functools

import jax
import jax.numpy as jnp
from jax import lax
from jax.experimental import pallas as pl
from jax.experimental.pallas import tpu as pltpu
from jax.experimental.pallas import tpu_sc as plsc

B, C, L = 4, 2048, 4096
GROUP_SIZES = (256, 256, 512, 1024)
GROUP_STARTS = (0, 256, 512, 1024)

NC, NS = 2, 16
NW = NC * NS
CH_PER_W = C // NW
CHUNK = 8
N_CHUNKS = CH_PER_W // CHUNK
N_ITER = B * N_CHUNKS

_mesh = plsc.VectorSubcoreMesh(core_axis_name="c", subcore_axis_name="s")


@functools.partial(
    pl.kernel,
    out_type=[jax.ShapeDtypeStruct((B, g, L), jnp.float32)
              for _ in range(3) for g in GROUP_SIZES],
    mesh=_mesh,
    scratch_types=[
        pltpu.VMEM((CH_PER_W,), jnp.int32),
        pltpu.VMEM((B * CH_PER_W,), jnp.int32),
        pltpu.VMEM((CHUNK, L), jnp.float32),
        pltpu.VMEM((CHUNK, L), jnp.float32),
        pltpu.VMEM((CHUNK, L), jnp.float32),
        pltpu.SemaphoreType.DMA,
        pltpu.SemaphoreType.DMA,
        pltpu.SemaphoreType.DMA,
        pltpu.SemaphoreType.DMA,
        pltpu.SemaphoreType.DMA,
        pltpu.SemaphoreType.DMA,
    ],
)
def _sc_regroup(q_hbm, k_hbm, v_hbm, idx_hbm,
                q0, q1, q2, q3, k0, k1, k2, k3, v0, v1, v2, v3,
                idx_v, idx_all, buf0, buf1, buf2,
                gsem0, gsem1, gsem2, wsem0, wsem1, wsem2):
    wid = lax.axis_index("s") * NC + lax.axis_index("c")
    base = wid * CH_PER_W

    pltpu.sync_copy(idx_hbm.at[pl.ds(base, CH_PER_W)], idx_v)
    for b in range(B):
        for j in range(CH_PER_W // 16):
            idx_all[pl.ds(b * CH_PER_W + j * 16, 16)] = (
                idx_v[pl.ds(j * 16, 16)] + b * C)

    bufs = (buf0, buf1, buf2)
    gsems = (gsem0, gsem1, gsem2)
    wsems = (wsem0, wsem1, wsem2)
    outs = ((q0, q1, q2, q3), (k0, k1, k2, k3), (v0, v1, v2, v3))
    tabs = (q_hbm, k_hbm, v_hbm)

    TOT = 3 * N_ITER

    def decode(i):
        if isinstance(i, int):
            return i // N_ITER, (i % N_ITER) // N_CHUNKS, i % N_CHUNKS
        t = i // N_ITER
        r = lax.rem(i, N_ITER)
        return t, r // N_CHUNKS, lax.rem(r, N_CHUNKS)

    for g in range(4):
        g_lo = GROUP_STARTS[g] // CH_PER_W
        g_hi = (GROUP_STARTS[g] + GROUP_SIZES[g]) // CH_PER_W

        @pl.when(jnp.logical_and(wid >= g_lo, wid < g_hi))
        def _():
            off0 = base - GROUP_STARTS[g]

            def start_gather(i, slot):
                t, b, c = decode(i)
                idx = idx_all.at[pl.ds(b * CH_PER_W + c * CHUNK, CHUNK)]
                if isinstance(i, int):
                    pltpu.async_copy(tabs[t].at[idx], bufs[slot],
                                     gsems[slot])
                    return
                for tt in range(3):
                    @pl.when(t == tt)
                    def _():
                        pltpu.async_copy(tabs[tt].at[idx], bufs[slot],
                                         gsems[slot])

            def wait_gather(slot):
                pltpu.make_async_copy(
                    tabs[0].at[pl.ds(0, CHUNK)], bufs[slot],
                    gsems[slot]).wait()

            def start_write(i, slot):
                t, b, c = decode(i)
                dst = (b, pl.ds(off0 + c * CHUNK, CHUNK))
                for tt in range(3):
                    @pl.when(t == tt)
                    def _():
                        pltpu.async_copy(bufs[slot],
                                         outs[tt][g].at[dst],
                                         wsems[slot])

            def wait_write(slot):
                pltpu.make_async_copy(
                    bufs[slot], outs[0][g].at[0, pl.ds(0, CHUNK)],
                    wsems[slot]).wait()

            for s in range(3):
                start_gather(s, s)

            def body(k, _):
                i = 3 * k
                for s in range(3):
                    wait_gather(s)
                    start_write(i + s, s)
                for s in range(3):
                    wait_write(s)

                    @pl.when(i + 3 + s < TOT)
                    def _():
                        start_gather(i + 3 + s, s)
                return 0

            lax.fori_loop(0, TOT // 3, body, 0)

    return None


CB = 256


def _mean_body(x_ref, o_ref):
    ys = []
    for b in range(B):
        xb = x_ref[b]
        p = xb[:, 0:128]
        for k in range(1, L // 128):
            p = xb[:, 128 * k:128 * k + 128] + p
        a = p[:, 0:8]
        for j in range(1, 16):
            a = p[:, 8 * j:8 * j + 8] + a
        b2 = a[:, 4:8] + a[:, 0:4]
        c2 = b2[:, 2:4] + b2[:, 0:2]
        s = c2[:, 1:2] + c2[:, 0:1]
        ys.append(s * jnp.float32(1.0 / L))
    o_ref[...] = (ys[1] + ys[3]) + (ys[0] + ys[2])


_mean_call = pl.pallas_call(
    _mean_body,
    grid=(C // CB,),
    in_specs=[pl.BlockSpec((B, CB, L), lambda i: (0, i, 0))],
    out_specs=pl.BlockSpec((CB, 1), lambda i: (i, 0)),
    out_shape=jax.ShapeDtypeStruct((C, 1), jnp.float32),
)


def _rank_body(zc_ref, zr_ref, o_ref):
    i = pl.program_id(0)
    zi = zc_ref[...]
    zj = zr_ref[...]
    ii = lax.broadcasted_iota(jnp.int32, (CB, C), 0) + i * CB
    jj = lax.broadcasted_iota(jnp.int32, (CB, C), 1)
    gt = (zj > zi).astype(jnp.int32)
    eq = (jnp.logical_and(zj == zi, jj < ii)).astype(jnp.int32)
    rank = jnp.sum(gt + eq, axis=1, keepdims=True)
    contrib = jnp.sum((rank == jj).astype(jnp.int32) * ii,
                      axis=0, keepdims=True)

    @pl.when(i == 0)
    def _():
        o_ref[...] = jnp.zeros((1, C), jnp.int32)

    o_ref[...] += contrib


_rank_call = pl.pallas_call(
    _rank_body,
    grid=(C // CB,),
    in_specs=[pl.BlockSpec((CB, 1), lambda i: (i, 0)),
              pl.BlockSpec((1, C), lambda i: (0, 0))],
    out_specs=pl.BlockSpec((1, C), lambda i: (0, 0)),
    out_shape=jax.ShapeDtypeStruct((1, C), jnp.int32),
)


def kernel(query, key, value):
    zc = _mean_call(query)
    sorted_indices = _rank_call(zc, zc.reshape(1, C)).reshape(C)

    qf = query.reshape(B * C, L)
    kf = key.reshape(B * C, L)
    vf = value.reshape(B * C, L)
    outs = _sc_regroup(qf, kf, vf, sorted_indices)
    return tuple(tuple(outs[t * 4:(t + 1) * 4]) for t in range(3))


# speedup vs baseline: 1.0062x; 1.0062x over previous
import functools

import jax
import jax.numpy as jnp
from jax import lax
from jax.experimental import pallas as pl
from jax.experimental.pallas import tpu as pltpu
from jax.experimental.pallas import tpu_sc as plsc

B, C, L = 4, 2048, 4096
GROUP_SIZES = (256, 256, 512, 1024)
GROUP_STARTS = (0, 256, 512, 1024)

NC, NS = 2, 16
NW = NC * NS
CH_PER_W = C // NW
CHUNK = 8
N_CHUNKS = CH_PER_W // CHUNK
N_ITER = B * N_CHUNKS

_mesh = plsc.VectorSubcoreMesh(core_axis_name="c", subcore_axis_name="s")


@functools.partial(
    pl.kernel,
    out_type=[jax.ShapeDtypeStruct((B, g, L), jnp.float32)
              for _ in range(3) for g in GROUP_SIZES],
    mesh=_mesh,
    scratch_types=[
        pltpu.VMEM((CH_PER_W,), jnp.int32),
        pltpu.VMEM((B * CH_PER_W,), jnp.int32),
        pltpu.VMEM((CHUNK, L), jnp.float32),
        pltpu.VMEM((CHUNK, L), jnp.float32),
        pltpu.VMEM((CHUNK, L), jnp.float32),
        pltpu.SemaphoreType.DMA,
        pltpu.SemaphoreType.DMA,
        pltpu.SemaphoreType.DMA,
        pltpu.SemaphoreType.DMA,
        pltpu.SemaphoreType.DMA,
        pltpu.SemaphoreType.DMA,
    ],
)
def _sc_regroup(q_hbm, k_hbm, v_hbm, idx_hbm,
                q0, q1, q2, q3, k0, k1, k2, k3, v0, v1, v2, v3,
                idx_v, idx_all, buf0, buf1, buf2,
                gsem0, gsem1, gsem2, wsem0, wsem1, wsem2):
    wid = lax.axis_index("s") * NC + lax.axis_index("c")
    base = wid * CH_PER_W

    pltpu.sync_copy(idx_hbm.at[pl.ds(base, CH_PER_W)], idx_v)
    for b in range(B):
        for j in range(CH_PER_W // 16):
            idx_all[pl.ds(b * CH_PER_W + j * 16, 16)] = (
                idx_v[pl.ds(j * 16, 16)] + b * C)

    bufs = (buf0, buf1, buf2)
    gsems = (gsem0, gsem1, gsem2)
    wsems = (wsem0, wsem1, wsem2)
    outs = ((q0, q1, q2, q3), (k0, k1, k2, k3), (v0, v1, v2, v3))
    tabs = (q_hbm, k_hbm, v_hbm)

    TOT = 3 * N_ITER

    def decode(i):
        if isinstance(i, int):
            return i // N_ITER, (i % N_ITER) // N_CHUNKS, i % N_CHUNKS
        t = i // N_ITER
        r = lax.rem(i, N_ITER)
        return t, r // N_CHUNKS, lax.rem(r, N_CHUNKS)

    for g in range(4):
        g_lo = GROUP_STARTS[g] // CH_PER_W
        g_hi = (GROUP_STARTS[g] + GROUP_SIZES[g]) // CH_PER_W

        @pl.when(jnp.logical_and(wid >= g_lo, wid < g_hi))
        def _():
            off0 = base - GROUP_STARTS[g]

            def start_gather(i, slot):
                t, b, c = decode(i)
                idx = idx_all.at[pl.ds(b * CH_PER_W + c * CHUNK, CHUNK)]
                if isinstance(i, int):
                    pltpu.async_copy(tabs[t].at[idx], bufs[slot],
                                     gsems[slot])
                    return
                for tt in range(3):
                    @pl.when(t == tt)
                    def _():
                        pltpu.async_copy(tabs[tt].at[idx], bufs[slot],
                                         gsems[slot])

            def wait_gather(slot):
                pltpu.make_async_copy(
                    tabs[0].at[pl.ds(0, CHUNK)], bufs[slot],
                    gsems[slot]).wait()

            def start_write(i, slot):
                t, b, c = decode(i)
                dst = (b, pl.ds(off0 + c * CHUNK, CHUNK))
                for tt in range(3):
                    @pl.when(t == tt)
                    def _():
                        pltpu.async_copy(bufs[slot],
                                         outs[tt][g].at[dst],
                                         wsems[slot])

            def wait_write(slot):
                pltpu.make_async_copy(
                    bufs[slot], outs[0][g].at[0, pl.ds(0, CHUNK)],
                    wsems[slot]).wait()

            for s in range(3):
                start_gather(s, s)

            def body(k, _):
                i = 3 * k
                for s in range(3):
                    wait_gather(s)
                    start_write(i + s, s)
                for s in range(3):
                    wait_write(s)

                    @pl.when(i + 3 + s < TOT)
                    def _():
                        start_gather(i + 3 + s, s)
                return 0

            lax.fori_loop(0, TOT // 3, body, 0)

    return None


CB = 256


def _mean_body(x_ref, o_ref):
    ys = []
    for b in range(B):
        xb = x_ref[b]
        p = xb[:, 0:128]
        for k in range(1, L // 128):
            p = xb[:, 128 * k:128 * k + 128] + p
        a = p[:, 0:8]
        for j in range(1, 16):
            a = p[:, 8 * j:8 * j + 8] + a
        b2 = a[:, 4:8] + a[:, 0:4]
        c2 = b2[:, 2:4] + b2[:, 0:2]
        s = c2[:, 1:2] + c2[:, 0:1]
        ys.append(s * jnp.float32(1.0 / L))
    o_ref[...] = (ys[1] + ys[3]) + (ys[0] + ys[2])


_mean_call = pl.pallas_call(
    _mean_body,
    grid=(C // CB,),
    in_specs=[pl.BlockSpec((B, CB, L), lambda i: (0, i, 0))],
    out_specs=pl.BlockSpec((CB, 1), lambda i: (i, 0)),
    out_shape=jax.ShapeDtypeStruct((C, 1), jnp.float32),
)


def _rank_body(zc_ref, zall_ref, o_ref):
    i = pl.program_id(0)
    zi = zc_ref[...]
    zj = lax.transpose(zall_ref[...], (1, 0))
    ii = lax.broadcasted_iota(jnp.int32, (CB, C), 0) + i * CB
    jj = lax.broadcasted_iota(jnp.int32, (CB, C), 1)
    gt = (zj > zi).astype(jnp.int32)
    eq = (jnp.logical_and(zj == zi, jj < ii)).astype(jnp.int32)
    rank = jnp.sum(gt + eq, axis=1, keepdims=True)
    contrib = jnp.sum((rank == jj).astype(jnp.int32) * ii,
                      axis=0)

    @pl.when(i == 0)
    def _():
        o_ref[...] = jnp.zeros((C,), jnp.int32)

    o_ref[...] += contrib


_rank_call = pl.pallas_call(
    _rank_body,
    grid=(C // CB,),
    in_specs=[pl.BlockSpec((CB, 1), lambda i: (i, 0)),
              pl.BlockSpec((C, 1), lambda i: (0, 0))],
    out_specs=pl.BlockSpec((C,), lambda i: (0,)),
    out_shape=jax.ShapeDtypeStruct((C,), jnp.int32),
)


def kernel(query, key, value):
    zc = _mean_call(query)
    sorted_indices = _rank_call(zc, zc)

    qf = query.reshape(B * C, L)
    kf = key.reshape(B * C, L)
    vf = value.reshape(B * C, L)
    outs = _sc_regroup(qf, kf, vf, sorted_indices)
    return tuple(tuple(outs[t * 4:(t + 1) * 4]) for t in range(3))


# speedup vs baseline: 1.0219x; 1.0156x over previous
import functools

import jax
import jax.numpy as jnp
from jax import lax
from jax.experimental import pallas as pl
from jax.experimental.pallas import tpu as pltpu
from jax.experimental.pallas import tpu_sc as plsc

B, C, L = 4, 2048, 4096
GROUP_SIZES = (256, 256, 512, 1024)
GROUP_STARTS = (0, 256, 512, 1024)

NC, NS = 2, 16
NW = NC * NS
CH_PER_W = C // NW
CHUNK = 8
NBUF = 3
N_CHUNKS = CH_PER_W // CHUNK
N_ITER = B * N_CHUNKS

_mesh = plsc.VectorSubcoreMesh(core_axis_name="c", subcore_axis_name="s")


@functools.partial(
    pl.kernel,
    out_type=[jax.ShapeDtypeStruct((B, g, L), jnp.float32)
              for _ in range(3) for g in GROUP_SIZES],
    mesh=_mesh,
    scratch_types=(
        [pltpu.VMEM((CH_PER_W,), jnp.int32),
         pltpu.VMEM((B * CH_PER_W,), jnp.int32)]
        + [pltpu.VMEM((CHUNK, L), jnp.float32) for _ in range(NBUF)]
        + [pltpu.SemaphoreType.DMA for _ in range(2 * NBUF)]
    ),
)
def _sc_regroup(q_hbm, k_hbm, v_hbm, idx_hbm,
                q0, q1, q2, q3, k0, k1, k2, k3, v0, v1, v2, v3,
                idx_v, idx_all, *bufs_sems):
    wid = lax.axis_index("s") * NC + lax.axis_index("c")
    base = wid * CH_PER_W

    pltpu.sync_copy(idx_hbm.at[pl.ds(base, CH_PER_W)], idx_v)
    for b in range(B):
        for j in range(CH_PER_W // 16):
            idx_all[pl.ds(b * CH_PER_W + j * 16, 16)] = (
                idx_v[pl.ds(j * 16, 16)] + b * C)

    bufs = bufs_sems[:NBUF]
    gsems = bufs_sems[NBUF:2 * NBUF]
    wsems = bufs_sems[2 * NBUF:]
    outs = ((q0, q1, q2, q3), (k0, k1, k2, k3), (v0, v1, v2, v3))
    tabs = (q_hbm, k_hbm, v_hbm)

    TOT = 3 * N_ITER

    def decode(i):
        if isinstance(i, int):
            return i // N_ITER, (i % N_ITER) // N_CHUNKS, i % N_CHUNKS
        t = i // N_ITER
        r = lax.rem(i, N_ITER)
        return t, r // N_CHUNKS, lax.rem(r, N_CHUNKS)

    for g in range(4):
        g_lo = GROUP_STARTS[g] // CH_PER_W
        g_hi = (GROUP_STARTS[g] + GROUP_SIZES[g]) // CH_PER_W

        @pl.when(jnp.logical_and(wid >= g_lo, wid < g_hi))
        def _():
            off0 = base - GROUP_STARTS[g]

            def start_gather(i, slot):
                t, b, c = decode(i)
                idx = idx_all.at[pl.ds(b * CH_PER_W + c * CHUNK, CHUNK)]
                if isinstance(i, int):
                    pltpu.async_copy(tabs[t].at[idx], bufs[slot],
                                     gsems[slot])
                    return
                for tt in range(3):
                    @pl.when(t == tt)
                    def _():
                        pltpu.async_copy(tabs[tt].at[idx], bufs[slot],
                                         gsems[slot])

            def wait_gather(slot):
                pltpu.make_async_copy(
                    tabs[0].at[pl.ds(0, CHUNK)], bufs[slot],
                    gsems[slot]).wait()

            def start_write(i, slot):
                t, b, c = decode(i)
                dst = (b, pl.ds(off0 + c * CHUNK, CHUNK))
                for tt in range(3):
                    @pl.when(t == tt)
                    def _():
                        pltpu.async_copy(bufs[slot],
                                         outs[tt][g].at[dst],
                                         wsems[slot])

            def wait_write(slot):
                pltpu.make_async_copy(
                    bufs[slot], outs[0][g].at[0, pl.ds(0, CHUNK)],
                    wsems[slot]).wait()

            for s in range(NBUF):
                start_gather(s, s)

            def body(k, _):
                i = NBUF * k
                for s in range(NBUF):
                    wait_gather(s)
                    start_write(i + s, s)
                for s in range(NBUF):
                    wait_write(s)

                    @pl.when(i + NBUF + s < TOT)
                    def _():
                        start_gather(i + NBUF + s, s)
                return 0

            lax.fori_loop(0, TOT // NBUF, body, 0)

    return None


CB = 256


def _mean_rank_body(x_ref, o_ref, z_scr):
    i = pl.program_id(0)
    ys = []
    for b in range(B):
        xb = x_ref[b]
        p = xb[:, 0:128]
        for k in range(1, L // 128):
            p = xb[:, 128 * k:128 * k + 128] + p
        a = p[:, 0:8]
        for j in range(1, 16):
            a = p[:, 8 * j:8 * j + 8] + a
        b2 = a[:, 4:8] + a[:, 0:4]
        c2 = b2[:, 2:4] + b2[:, 0:2]
        s = c2[:, 1:2] + c2[:, 0:1]
        ys.append(s * jnp.float32(1.0 / L))
    z_scr[pl.ds(i * CB, CB), :] = (ys[1] + ys[3]) + (ys[0] + ys[2])

    @pl.when(i == C // CB - 1)
    def _():
        zrow = lax.transpose(z_scr[...], (1, 0))

        def rbody(k, _):
            zi = z_scr[pl.ds(k * CB, CB), :]
            ii = lax.broadcasted_iota(jnp.int32, (CB, C), 0) + k * CB
            jj = lax.broadcasted_iota(jnp.int32, (CB, C), 1)
            gt = (zrow > zi).astype(jnp.int32)
            eq = (jnp.logical_and(zrow == zi, jj < ii)).astype(jnp.int32)
            rank = jnp.sum(gt + eq, axis=1, keepdims=True)
            contrib = jnp.sum((rank == jj).astype(jnp.int32) * ii, axis=0)

            @pl.when(k == 0)
            def _():
                o_ref[...] = contrib

            @pl.when(k > 0)
            def _():
                o_ref[...] += contrib

            return 0

        lax.fori_loop(0, C // CB, rbody, 0)


_mean_rank_call = pl.pallas_call(
    _mean_rank_body,
    grid=(C // CB,),
    in_specs=[pl.BlockSpec((B, CB, L), lambda i: (0, i, 0))],
    out_specs=pl.BlockSpec((C,), lambda i: (0,)),
    out_shape=jax.ShapeDtypeStruct((C,), jnp.int32),
    scratch_shapes=[pltpu.VMEM((C, 1), jnp.float32)],
)


def kernel(query, key, value):
    sorted_indices = _mean_rank_call(query)

    qf = query.reshape(B * C, L)
    kf = key.reshape(B * C, L)
    vf = value.reshape(B * C, L)
    outs = _sc_regroup(qf, kf, vf, sorted_indices)
    return tuple(tuple(outs[t * 4:(t + 1) * 4]) for t in range(3))


# speedup vs baseline: 1.0256x; 1.0036x over previous
import functools

import jax
import jax.numpy as jnp
from jax import lax
from jax.experimental import pallas as pl
from jax.experimental.pallas import tpu as pltpu
from jax.experimental.pallas import tpu_sc as plsc

B, C, L = 4, 2048, 4096
GROUP_SIZES = (256, 256, 512, 1024)
GROUP_STARTS = (0, 256, 512, 1024)

NC, NS = 2, 16
NW = NC * NS
CH_PER_W = C // NW
CHUNK = 8
NBUF = 3
N_CHUNKS = CH_PER_W // CHUNK
N_ITER = B * N_CHUNKS

_mesh = plsc.VectorSubcoreMesh(core_axis_name="c", subcore_axis_name="s")


@functools.partial(
    pl.kernel,
    out_type=[jax.ShapeDtypeStruct((B, g, L), jnp.float32)
              for _ in range(3) for g in GROUP_SIZES],
    mesh=_mesh,
    scratch_types=(
        [pltpu.VMEM((CH_PER_W,), jnp.int32),
         pltpu.VMEM((B * CH_PER_W,), jnp.int32)]
        + [pltpu.VMEM((CHUNK, L), jnp.float32) for _ in range(NBUF)]
        + [pltpu.SemaphoreType.DMA for _ in range(2 * NBUF)]
    ),
)
def _sc_regroup(q_hbm, k_hbm, v_hbm, idx_hbm,
                q0, q1, q2, q3, k0, k1, k2, k3, v0, v1, v2, v3,
                idx_v, idx_all, *bufs_sems):
    wid = lax.axis_index("s") * NC + lax.axis_index("c")
    base = wid * CH_PER_W

    pltpu.sync_copy(idx_hbm.at[pl.ds(base, CH_PER_W)], idx_v)
    for b in range(B):
        for j in range(CH_PER_W // 16):
            idx_all[pl.ds(b * CH_PER_W + j * 16, 16)] = (
                idx_v[pl.ds(j * 16, 16)] + b * C)

    bufs = bufs_sems[:NBUF]
    gsems = bufs_sems[NBUF:2 * NBUF]
    wsems = bufs_sems[2 * NBUF:]
    outs = ((q0, q1, q2, q3), (k0, k1, k2, k3), (v0, v1, v2, v3))
    tabs = (q_hbm, k_hbm, v_hbm)

    TOT = 3 * N_ITER

    def decode(i):
        if isinstance(i, int):
            return i // N_ITER, (i % N_ITER) // N_CHUNKS, i % N_CHUNKS
        t = i // N_ITER
        r = lax.rem(i, N_ITER)
        return t, r // N_CHUNKS, lax.rem(r, N_CHUNKS)

    for g in range(4):
        g_lo = GROUP_STARTS[g] // CH_PER_W
        g_hi = (GROUP_STARTS[g] + GROUP_SIZES[g]) // CH_PER_W

        @pl.when(jnp.logical_and(wid >= g_lo, wid < g_hi))
        def _():
            off0 = base - GROUP_STARTS[g]

            def start_gather(i, slot):
                t, b, c = decode(i)
                idx = idx_all.at[pl.ds(b * CH_PER_W + c * CHUNK, CHUNK)]
                if isinstance(i, int):
                    pltpu.async_copy(tabs[t].at[idx], bufs[slot],
                                     gsems[slot])
                    return
                for tt in range(3):
                    @pl.when(t == tt)
                    def _():
                        pltpu.async_copy(tabs[tt].at[idx], bufs[slot],
                                         gsems[slot])

            def wait_gather(slot):
                pltpu.make_async_copy(
                    tabs[0].at[pl.ds(0, CHUNK)], bufs[slot],
                    gsems[slot]).wait()

            def start_write(i, slot):
                t, b, c = decode(i)
                dst = (b, pl.ds(off0 + c * CHUNK, CHUNK))
                for tt in range(3):
                    @pl.when(t == tt)
                    def _():
                        pltpu.async_copy(bufs[slot],
                                         outs[tt][g].at[dst],
                                         wsems[slot])

            def wait_write(slot):
                pltpu.make_async_copy(
                    bufs[slot], outs[0][g].at[0, pl.ds(0, CHUNK)],
                    wsems[slot]).wait()

            for s in range(NBUF):
                start_gather(s, s)

            def body(k, _):
                i = NBUF * k
                for s in range(NBUF):
                    wait_gather(s)
                    start_write(i + s, s)
                for s in range(NBUF):
                    wait_write(s)

                    @pl.when(i + NBUF + s < TOT)
                    def _():
                        start_gather(i + NBUF + s, s)
                return 0

            lax.fori_loop(0, TOT // NBUF, body, 0)

    return None


CB = 128


def _mean_rank_body(x_ref, o_ref, z_scr):
    i = pl.program_id(0)
    ys = []
    for b in range(B):
        xb = x_ref[b]
        p = xb[:, 0:128]
        for k in range(1, L // 128):
            p = xb[:, 128 * k:128 * k + 128] + p
        a = p[:, 0:8]
        for j in range(1, 16):
            a = p[:, 8 * j:8 * j + 8] + a
        b2 = a[:, 4:8] + a[:, 0:4]
        c2 = b2[:, 2:4] + b2[:, 0:2]
        s = c2[:, 1:2] + c2[:, 0:1]
        ys.append(s * jnp.float32(1.0 / L))
    z_scr[pl.ds(i * CB, CB), :] = (ys[1] + ys[3]) + (ys[0] + ys[2])

    @pl.when(i == C // CB - 1)
    def _():
        zrow = lax.transpose(z_scr[...], (1, 0))

        def rbody(k, _):
            zi = z_scr[pl.ds(k * CB, CB), :]
            ii = lax.broadcasted_iota(jnp.int32, (CB, C), 0) + k * CB
            jj = lax.broadcasted_iota(jnp.int32, (CB, C), 1)
            gt = (zrow > zi).astype(jnp.int32)
            eq = (jnp.logical_and(zrow == zi, jj < ii)).astype(jnp.int32)
            rank = jnp.sum(gt + eq, axis=1, keepdims=True)
            contrib = jnp.sum((rank == jj).astype(jnp.int32) * ii, axis=0)

            @pl.when(k == 0)
            def _():
                o_ref[...] = contrib

            @pl.when(k > 0)
            def _():
                o_ref[...] += contrib

            return 0

        lax.fori_loop(0, C // CB, rbody, 0)


_mean_rank_call = pl.pallas_call(
    _mean_rank_body,
    grid=(C // CB,),
    in_specs=[pl.BlockSpec((B, CB, L), lambda i: (0, i, 0))],
    out_specs=pl.BlockSpec((C,), lambda i: (0,)),
    out_shape=jax.ShapeDtypeStruct((C,), jnp.int32),
    scratch_shapes=[pltpu.VMEM((C, 1), jnp.float32)],
)


def kernel(query, key, value):
    sorted_indices = _mean_rank_call(query)

    qf = query.reshape(B * C, L)
    kf = key.reshape(B * C, L)
    vf = value.reshape(B * C, L)
    outs = _sc_regroup(qf, kf, vf, sorted_indices)
    return tuple(tuple(outs[t * 4:(t + 1) * 4]) for t in range(3))
